# baseline (speedup 1.0000x reference)
import functools

import jax
import jax.numpy as jnp
from jax import lax
from jax.experimental import pallas as pl
from jax.experimental.pallas import tpu as pltpu
from jax.experimental.pallas import tpu_sc as plsc

N_PAD = 5120
PER_W = N_PAD // 16
VREGS = PER_W // 16
ALL_VREGS = N_PAD // 16
MAX_OUT = 100


def _splat(x):
    return jnp.full((16,), x)


def _vperm(v, p):
    return lax.gather(
        v, p[:, None],
        dimension_numbers=lax.GatherDimensionNumbers(
            offset_dims=(), collapsed_slice_dims=(0,), start_index_map=(0,)),
        slice_sizes=(1,),
        mode=lax.GatherScatterMode.PROMISE_IN_BOUNDS)


def _amax_merge(m, mi, b, bi):
    take = jnp.logical_or(b > m, jnp.logical_and(b == m, bi < mi))
    return jnp.where(take, b, m), jnp.where(take, bi, mi)


def _nms_body(y1h, x1h, y2h, x2h, sh, outh,
              y1v, x1v, y2v, x2v, areav, sv,
              stage, table_sh, tablev, outv):
    cid = lax.axis_index("c")
    wid = lax.axis_index("s")
    base = wid * PER_W
    iota = lax.iota(jnp.int32, 16)

    pltpu.sync_copy(y1h, y1v)
    pltpu.sync_copy(x1h, x1v)
    pltpu.sync_copy(y2h, y2v)
    pltpu.sync_copy(x2h, x2v)
    pltpu.sync_copy(sh.at[pl.ds(base, PER_W)], sv)

    for j in range(ALL_VREGS):
        sl = pl.ds(j * 16, 16)
        areav[sl] = (y2v[sl] - y1v[sl]) * (x2v[sl] - x1v[sl])

    scores0 = [sv[pl.ds(j * 16, 16)] for j in range(VREGS)]

    perms = [jnp.bitwise_xor(iota, jnp.int32(p)) for p in (1, 2, 4, 8)]

    zero = jnp.zeros((16,), jnp.float32)

    def body(t, carry):
        py1, px1, py2, px2, pa = carry[:5]
        scores = carry[5:]

        best = jnp.full((16,), -2.0)
        bidx = jnp.zeros((16,), jnp.int32)
        idxv = base + iota
        new_scores = []
        for j in range(VREGS):
            sl = pl.ds(base + j * 16, 16)
            iy1 = jnp.maximum(py1, y1v[sl])
            ix1 = jnp.maximum(px1, x1v[sl])
            iy2 = jnp.minimum(py2, y2v[sl])
            ix2 = jnp.minimum(px2, x2v[sl])
            inter = jnp.maximum(iy2 - iy1, 0.0) * jnp.maximum(ix2 - ix1, 0.0)
            union = pa + areav[sl] - inter
            s = jnp.where(inter + inter > union, -1.0, scores[j])
            new_scores.append(s)
            gt = s > best
            best = jnp.where(gt, s, best)
            bidx = jnp.where(gt, idxv, bidx)
            idxv = idxv + 16

        stage[pl.ds(0, 16)] = best
        stage[pl.ds(16, 16)] = plsc.bitcast(bidx, jnp.float32)
        off = (t & 1) * (16 * 32)
        pltpu.sync_copy(stage, table_sh.at[pl.ds(off + wid * 32, 32)])
        plsc.subcore_barrier()
        pltpu.sync_copy(table_sh.at[pl.ds(off, 16 * 32)], tablev)

        ms = [tablev[pl.ds(w * 32, 16)] for w in range(16)]
        mis = [plsc.bitcast(tablev[pl.ds(w * 32 + 16, 16)], jnp.int32)
               for w in range(16)]
        width = 16
        while width > 1:
            width //= 2
            for w in range(width):
                ms[w], mis[w] = _amax_merge(ms[w], mis[w],
                                            ms[w + width], mis[w + width])
        m, mi = ms[0], mis[0]
        for p in perms:
            m2 = _vperm(m, p)
            mi2 = _vperm(mi, p)
            m, mi = _amax_merge(m, mi, m2, mi2)

        npy1 = plsc.load_gather(y1v, [mi])
        npx1 = plsc.load_gather(x1v, [mi])
        npy2 = plsc.load_gather(y2v, [mi])
        npx2 = plsc.load_gather(x2v, [mi])
        npa = plsc.load_gather(areav, [mi])

        hasf = (m >= 0.0).astype(jnp.float32)

        @pl.when(jnp.logical_and(cid == 0, wid == 0))
        def _():
            v = jnp.where(iota == 0, npy1,
                jnp.where(iota == 1, npx1,
                jnp.where(iota == 2, npy2, npx2))) * hasf
            plsc.store_scatter(outv, [t * 4 + iota], v, mask=iota < 4)

        return (npy1, npx1, npy2, npx2, npa, *new_scores)

    lax.fori_loop(0, MAX_OUT, body, (zero, zero, zero, zero, zero, *scores0),
                  unroll=False)

    @pl.when(jnp.logical_and(cid == 0, wid == 0))
    def _():
        pltpu.sync_copy(outv.at[pl.ds(0, MAX_OUT * 4)], outh)


@jax.jit
def _nms(y1, x1, y2, x2, s):
    mesh = plsc.VectorSubcoreMesh(core_axis_name="c", subcore_axis_name="s")
    f = functools.partial(
        pl.kernel,
        mesh=mesh,
        compiler_params=pltpu.CompilerParams(needs_layout_passes=False),
        out_type=jax.ShapeDtypeStruct((MAX_OUT * 4,), jnp.float32),
        scratch_types=[
            pltpu.VMEM((N_PAD,), jnp.float32),
            pltpu.VMEM((N_PAD,), jnp.float32),
            pltpu.VMEM((N_PAD,), jnp.float32),
            pltpu.VMEM((N_PAD,), jnp.float32),
            pltpu.VMEM((N_PAD,), jnp.float32),
            pltpu.VMEM((PER_W,), jnp.float32),
            pltpu.VMEM((32,), jnp.float32),
            pltpu.VMEM_SHARED((2 * 16 * 32,), jnp.float32),
            pltpu.VMEM((16 * 32,), jnp.float32),
            pltpu.VMEM((MAX_OUT * 4 + 16,), jnp.float32),
        ],
    )(_nms_body)
    return f(y1, x1, y2, x2, s)


def kernel(boxes, scores, max_output_size):
    n = boxes.shape[0]
    pad = N_PAD - n
    y1 = jnp.pad(boxes[:, 0], (0, pad))
    x1 = jnp.pad(boxes[:, 1], (0, pad))
    y2 = jnp.pad(boxes[:, 2], (0, pad))
    x2 = jnp.pad(boxes[:, 3], (0, pad))
    s = jnp.pad(scores, (0, pad), constant_values=-1.0)
    out = _nms(y1, x1, y2, x2, s).reshape(MAX_OUT, 4)
    keep = (lax.iota(jnp.int32, MAX_OUT) < max_output_size)[:, None]
    return jnp.where(keep, out, 0.0)


# speedup vs baseline: 1.0361x; 1.0361x over previous
import functools

import jax
import jax.numpy as jnp
from jax import lax
from jax.experimental import pallas as pl
from jax.experimental.pallas import tpu as pltpu
from jax.experimental.pallas import tpu_sc as plsc

N_PAD = 5120
PER_W = N_PAD // 16
VREGS = PER_W // 16
ALL_VREGS = N_PAD // 16
MAX_OUT = 100


def _splat(x):
    return jnp.full((16,), x)


def _vperm(v, p):
    return lax.gather(
        v, p[:, None],
        dimension_numbers=lax.GatherDimensionNumbers(
            offset_dims=(), collapsed_slice_dims=(0,), start_index_map=(0,)),
        slice_sizes=(1,),
        mode=lax.GatherScatterMode.PROMISE_IN_BOUNDS)


def _amax_merge(m, mi, b, bi):
    take = jnp.logical_or(b > m, jnp.logical_and(b == m, bi < mi))
    return jnp.where(take, b, m), jnp.where(take, bi, mi)


def _nms_body(y1h, x1h, y2h, x2h, sh, outh,
              y1v, x1v, y2v, x2v, areav, sv,
              stage, table_sh, tablev, outv):
    cid = lax.axis_index("c")
    wid = lax.axis_index("s")
    base = wid * PER_W
    iota = lax.iota(jnp.int32, 16)

    pltpu.sync_copy(y1h, y1v)
    pltpu.sync_copy(x1h, x1v)
    pltpu.sync_copy(y2h, y2v)
    pltpu.sync_copy(x2h, x2v)
    pltpu.sync_copy(sh.at[pl.ds(base, PER_W)], sv)

    for j in range(ALL_VREGS):
        sl = pl.ds(j * 16, 16)
        areav[sl] = (y2v[sl] - y1v[sl]) * (x2v[sl] - x1v[sl])

    scores0 = [sv[pl.ds(j * 16, 16)] for j in range(VREGS)]

    perms = [jnp.bitwise_xor(iota, jnp.int32(p)) for p in (1, 2, 4, 8)]

    zero = jnp.zeros((16,), jnp.float32)

    def body(t, carry):
        py1, px1, py2, px2, pa = carry[:5]
        scores = carry[5:]

        best = jnp.full((16,), -2.0)
        bidx = jnp.zeros((16,), jnp.int32)
        idxv = base + iota
        new_scores = []
        for j in range(VREGS):
            sl = pl.ds(base + j * 16, 16)
            iy1 = jnp.maximum(py1, y1v[sl])
            ix1 = jnp.maximum(px1, x1v[sl])
            iy2 = jnp.minimum(py2, y2v[sl])
            ix2 = jnp.minimum(px2, x2v[sl])
            inter = jnp.maximum(iy2 - iy1, 0.0) * jnp.maximum(ix2 - ix1, 0.0)
            union = pa + areav[sl] - inter
            s = jnp.where(inter + inter > union, -1.0, scores[j])
            new_scores.append(s)
            gt = s > best
            best = jnp.where(gt, s, best)
            bidx = jnp.where(gt, idxv, bidx)
            idxv = idxv + 16

        stage[pl.ds(0, 16)] = best
        stage[pl.ds(16, 16)] = plsc.bitcast(bidx, jnp.float32)
        off = (t & 1) * (16 * 32)
        pltpu.sync_copy(stage, table_sh.at[pl.ds(off + wid * 32, 32)])
        plsc.subcore_barrier()
        pltpu.sync_copy(table_sh.at[pl.ds(off, 16 * 32)], tablev)

        ms = [tablev[pl.ds(w * 32, 16)] for w in range(16)]
        mis = [plsc.bitcast(tablev[pl.ds(w * 32 + 16, 16)], jnp.int32)
               for w in range(16)]
        width = 16
        while width > 1:
            width //= 2
            for w in range(width):
                ms[w], mis[w] = _amax_merge(ms[w], mis[w],
                                            ms[w + width], mis[w + width])
        m, mi = ms[0], mis[0]
        for p in perms:
            m2 = _vperm(m, p)
            mi2 = _vperm(mi, p)
            m, mi = _amax_merge(m, mi, m2, mi2)

        npy1 = plsc.load_gather(y1v, [mi])
        npx1 = plsc.load_gather(x1v, [mi])
        npy2 = plsc.load_gather(y2v, [mi])
        npx2 = plsc.load_gather(x2v, [mi])
        npa = plsc.load_gather(areav, [mi])

        hasf = (m >= 0.0).astype(jnp.float32)

        @pl.when(jnp.logical_and(cid == 0, wid == 0))
        def _():
            v = jnp.where(iota == 0, npy1,
                jnp.where(iota == 1, npx1,
                jnp.where(iota == 2, npy2, npx2))) * hasf
            plsc.store_scatter(outv, [t * 4 + iota], v, mask=iota < 4)

        return (npy1, npx1, npy2, npx2, npa, *new_scores)

    lax.fori_loop(0, MAX_OUT, body, (zero, zero, zero, zero, zero, *scores0),
                  unroll=False)

    @pl.when(jnp.logical_and(cid == 0, wid == 0))
    def _():
        pltpu.sync_copy(outv.at[pl.ds(0, MAX_OUT * 4)], outh)


@jax.jit
def _nms(y1, x1, y2, x2, s):
    mesh = plsc.VectorSubcoreMesh(core_axis_name="c", subcore_axis_name="s", num_cores=1)
    f = functools.partial(
        pl.kernel,
        mesh=mesh,
        compiler_params=pltpu.CompilerParams(needs_layout_passes=False),
        out_type=jax.ShapeDtypeStruct((MAX_OUT * 4,), jnp.float32),
        scratch_types=[
            pltpu.VMEM((N_PAD,), jnp.float32),
            pltpu.VMEM((N_PAD,), jnp.float32),
            pltpu.VMEM((N_PAD,), jnp.float32),
            pltpu.VMEM((N_PAD,), jnp.float32),
            pltpu.VMEM((N_PAD,), jnp.float32),
            pltpu.VMEM((PER_W,), jnp.float32),
            pltpu.VMEM((32,), jnp.float32),
            pltpu.VMEM_SHARED((2 * 16 * 32,), jnp.float32),
            pltpu.VMEM((16 * 32,), jnp.float32),
            pltpu.VMEM((MAX_OUT * 4 + 16,), jnp.float32),
        ],
    )(_nms_body)
    return f(y1, x1, y2, x2, s)


def kernel(boxes, scores, max_output_size):
    n = boxes.shape[0]
    pad = N_PAD - n
    y1 = jnp.pad(boxes[:, 0], (0, pad))
    x1 = jnp.pad(boxes[:, 1], (0, pad))
    y2 = jnp.pad(boxes[:, 2], (0, pad))
    x2 = jnp.pad(boxes[:, 3], (0, pad))
    s = jnp.pad(scores, (0, pad), constant_values=-1.0)
    out = _nms(y1, x1, y2, x2, s).reshape(MAX_OUT, 4)
    keep = (lax.iota(jnp.int32, MAX_OUT) < max_output_size)[:, None]
    return jnp.where(keep, out, 0.0)


# speedup vs baseline: 1.0465x; 1.0101x over previous
import functools

import jax
import jax.numpy as jnp
from jax import lax
from jax.experimental import pallas as pl
from jax.experimental.pallas import tpu as pltpu
from jax.experimental.pallas import tpu_sc as plsc

N_REAL = 5000
N_PAD = 5120
PER_W = N_PAD // 16
VREGS = PER_W // 16
LAST_REAL = N_REAL - 15 * PER_W
MAX_OUT = 100


def _splat(x):
    return jnp.full((16,), x)


def _vperm(v, p):
    return lax.gather(
        v, p[:, None],
        dimension_numbers=lax.GatherDimensionNumbers(
            offset_dims=(), collapsed_slice_dims=(0,), start_index_map=(0,)),
        slice_sizes=(1,),
        mode=lax.GatherScatterMode.PROMISE_IN_BOUNDS)


def _amax_merge(m, mi, b, bi):
    take = jnp.logical_or(b > m, jnp.logical_and(b == m, bi < mi))
    return jnp.where(take, b, m), jnp.where(take, bi, mi)


def _nms_body(bh, sh, outh, boxesv, sv, areav, stage, table_sh, tablev, outv,
              sem):
    cid = lax.axis_index("c")
    wid = lax.axis_index("s")
    base = wid * PER_W
    iota = lax.iota(jnp.int32, 16)

    hbox = pltpu.async_copy(bh, boxesv.at[pl.ds(0, N_REAL * 4)], sem)

    @pl.when(wid < 15)
    def _():
        pltpu.sync_copy(sh.at[pl.ds(base, PER_W)], sv)

    @pl.when(wid == 15)
    def _():
        pltpu.sync_copy(sh.at[pl.ds(15 * PER_W, LAST_REAL)],
                        sv.at[pl.ds(0, LAST_REAL)])
        for k in range((PER_W - LAST_REAL) // 16):
            sv[pl.ds(LAST_REAL + k * 16, 16)] = jnp.full((16,), -1.0)

    hbox.wait()

    for k in range((N_PAD - N_REAL) * 4 // 16):
        boxesv[pl.ds(N_REAL * 4 + k * 16, 16)] = jnp.zeros((16,), jnp.float32)

    for j in range(VREGS):
        f0 = (base + j * 16 + iota) * 4
        by1 = plsc.load_gather(boxesv, [f0])
        bx1 = plsc.load_gather(boxesv, [f0 + 1])
        by2 = plsc.load_gather(boxesv, [f0 + 2])
        bx2 = plsc.load_gather(boxesv, [f0 + 3])
        areav[pl.ds(j * 16, 16)] = (by2 - by1) * (bx2 - bx1)

    scores0 = [sv[pl.ds(j * 16, 16)] for j in range(VREGS)]

    perms = [jnp.bitwise_xor(iota, jnp.int32(p)) for p in (1, 2, 4, 8)]

    zero = jnp.zeros((16,), jnp.float32)

    def body(t, carry):
        py1, px1, py2, px2, pa = carry[:5]
        scores = carry[5:]

        best = jnp.full((16,), -2.0)
        bidx = jnp.zeros((16,), jnp.int32)
        idxv = base + iota
        idxv4 = idxv * 4
        new_scores = []
        for j in range(VREGS):
            by1 = plsc.load_gather(boxesv, [idxv4])
            bx1 = plsc.load_gather(boxesv, [idxv4 + 1])
            by2 = plsc.load_gather(boxesv, [idxv4 + 2])
            bx2 = plsc.load_gather(boxesv, [idxv4 + 3])
            iy1 = jnp.maximum(py1, by1)
            ix1 = jnp.maximum(px1, bx1)
            iy2 = jnp.minimum(py2, by2)
            ix2 = jnp.minimum(px2, bx2)
            inter = jnp.maximum(iy2 - iy1, 0.0) * jnp.maximum(ix2 - ix1, 0.0)
            union = pa + areav[pl.ds(j * 16, 16)] - inter
            s = jnp.where(inter + inter > union, -1.0, scores[j])
            new_scores.append(s)
            gt = s > best
            best = jnp.where(gt, s, best)
            bidx = jnp.where(gt, idxv, bidx)
            idxv = idxv + 16
            idxv4 = idxv4 + 64

        stage[pl.ds(0, 16)] = best
        stage[pl.ds(16, 16)] = plsc.bitcast(bidx, jnp.float32)
        off = (t & 1) * (16 * 32)
        pltpu.sync_copy(stage, table_sh.at[pl.ds(off + wid * 32, 32)])
        plsc.subcore_barrier()
        pltpu.sync_copy(table_sh.at[pl.ds(off, 16 * 32)], tablev)

        ms = [tablev[pl.ds(w * 32, 16)] for w in range(16)]
        mis = [plsc.bitcast(tablev[pl.ds(w * 32 + 16, 16)], jnp.int32)
               for w in range(16)]
        width = 16
        while width > 1:
            width //= 2
            for w in range(width):
                ms[w], mis[w] = _amax_merge(ms[w], mis[w],
                                            ms[w + width], mis[w + width])
        m, mi = ms[0], mis[0]
        for p in perms:
            m, mi = _amax_merge(m, mi, _vperm(m, p), _vperm(mi, p))

        mi4 = mi * 4
        npy1 = plsc.load_gather(boxesv, [mi4])
        npx1 = plsc.load_gather(boxesv, [mi4 + 1])
        npy2 = plsc.load_gather(boxesv, [mi4 + 2])
        npx2 = plsc.load_gather(boxesv, [mi4 + 3])
        npa = (npy2 - npy1) * (npx2 - npx1)

        hasf = (m >= 0.0).astype(jnp.float32)

        @pl.when(jnp.logical_and(cid == 0, wid == 0))
        def _():
            v = jnp.where(iota == 0, npy1,
                jnp.where(iota == 1, npx1,
                jnp.where(iota == 2, npy2, npx2))) * hasf
            plsc.store_scatter(outv, [t * 4 + iota], v, mask=iota < 4)

        return (npy1, npx1, npy2, npx2, npa, *new_scores)

    lax.fori_loop(0, MAX_OUT, body, (zero, zero, zero, zero, zero, *scores0),
                  unroll=False)

    @pl.when(jnp.logical_and(cid == 0, wid == 0))
    def _():
        pltpu.sync_copy(outv.at[pl.ds(0, MAX_OUT * 4)], outh)


@jax.jit
def _nms(boxes, scores):
    mesh = plsc.VectorSubcoreMesh(core_axis_name="c", subcore_axis_name="s",
                                  num_cores=1)
    f = functools.partial(
        pl.kernel,
        mesh=mesh,
        compiler_params=pltpu.CompilerParams(needs_layout_passes=False),
        out_type=jax.ShapeDtypeStruct((MAX_OUT * 4,), jnp.float32),
        scratch_types=[
            pltpu.VMEM((N_PAD * 4,), jnp.float32),
            pltpu.VMEM((PER_W,), jnp.float32),
            pltpu.VMEM((PER_W,), jnp.float32),
            pltpu.VMEM((32,), jnp.float32),
            pltpu.VMEM_SHARED((2 * 16 * 32,), jnp.float32),
            pltpu.VMEM((16 * 32,), jnp.float32),
            pltpu.VMEM((MAX_OUT * 4 + 16,), jnp.float32),
            pltpu.SemaphoreType.DMA,
        ],
    )(_nms_body)
    return f(boxes.reshape(-1), scores)


def kernel(boxes, scores, max_output_size):
    out = _nms(boxes, scores).reshape(MAX_OUT, 4)
    keep = (lax.iota(jnp.int32, MAX_OUT) < max_output_size)[:, None]
    return jnp.where(keep, out, 0.0)


# speedup vs baseline: 1.1725x; 1.1204x over previous
import functools

import jax
import jax.numpy as jnp
from jax import lax
from jax.experimental import pallas as pl
from jax.experimental.pallas import tpu as pltpu
from jax.experimental.pallas import tpu_sc as plsc

N_PAD = 5120
PER_W = N_PAD // 16
VREGS = PER_W // 16
MAX_OUT = 100
ROW = 8


def _splat(x):
    return jnp.full((16,), x)


def _nms_body(y1h, x1h, y2h, x2h, sh, outh,
              y1v, x1v, y2v, x2v, sv, areav,
              stage, table_sh, tablev, outv):
    cid = lax.axis_index("c")
    wid = lax.axis_index("s")
    base = wid * PER_W
    iota = lax.iota(jnp.int32, 16)

    pltpu.sync_copy(y1h.at[pl.ds(base, PER_W)], y1v)
    pltpu.sync_copy(x1h.at[pl.ds(base, PER_W)], x1v)
    pltpu.sync_copy(y2h.at[pl.ds(base, PER_W)], y2v)
    pltpu.sync_copy(x2h.at[pl.ds(base, PER_W)], x2v)
    pltpu.sync_copy(sh.at[pl.ds(base, PER_W)], sv)

    scores0 = []
    for j in range(VREGS):
        sl = pl.ds(j * 16, 16)
        areav[sl] = (y2v[sl] - y1v[sl]) * (x2v[sl] - x1v[sl])
        scores0.append(sv[sl])

    zero = jnp.zeros((16,), jnp.float32)

    def body(t, carry):
        py1, px1, py2, px2, pa = carry[:5]
        scores = carry[5:]

        best = jnp.full((16,), -2.0)
        bidx = jnp.zeros((16,), jnp.int32)
        idxv = base + iota
        new_scores = []
        for j in range(VREGS):
            sl = pl.ds(j * 16, 16)
            iy1 = jnp.maximum(py1, y1v[sl])
            ix1 = jnp.maximum(px1, x1v[sl])
            iy2 = jnp.minimum(py2, y2v[sl])
            ix2 = jnp.minimum(px2, x2v[sl])
            inter = jnp.maximum(iy2 - iy1, 0.0) * jnp.maximum(ix2 - ix1, 0.0)
            union = pa + areav[sl] - inter
            s = jnp.where(inter + inter > union, -1.0, scores[j])
            new_scores.append(s)
            gt = s > best
            best = jnp.where(gt, s, best)
            bidx = jnp.where(gt, idxv, bidx)
            idxv = idxv + 16

        lmax = jnp.max(best)
        lidx = jnp.min(jnp.where(best == lmax, bidx, jnp.int32(1 << 30)))

        li = _splat(lidx - base)
        wy1 = plsc.load_gather(y1v, [li])
        wx1 = plsc.load_gather(x1v, [li])
        wy2 = plsc.load_gather(y2v, [li])
        wx2 = plsc.load_gather(x2v, [li])

        gsp = _splat(lmax)
        lsp = _splat(lidx.astype(jnp.float32))
        row = jnp.where(iota == 0, gsp,
              jnp.where(iota == 1, lsp,
              jnp.where(iota == 2, wy1,
              jnp.where(iota == 3, wx1,
              jnp.where(iota == 4, wy2, wx2)))))
        stage[...] = row
        off = (t & 1) * (16 * ROW)
        pltpu.sync_copy(stage.at[pl.ds(0, ROW)],
                        table_sh.at[pl.ds(off + wid * ROW, ROW)])
        plsc.subcore_barrier()
        pltpu.sync_copy(table_sh.at[pl.ds(off, 16 * ROW)], tablev)

        col = iota * ROW
        vals = plsc.load_gather(tablev, [col])
        gidx = plsc.load_gather(tablev, [col + 1])
        gmax = jnp.max(vals)
        widf = jnp.min(jnp.where(vals == gmax, gidx, jnp.float32(1e9)))
        rowm = jnp.logical_and(vals == gmax, gidx == widf)
        wrow = jnp.min(jnp.where(rowm, iota, jnp.int32(999)))
        rb = wrow * ROW
        npy1 = plsc.load_gather(tablev, [_splat(rb + 2)])
        npx1 = plsc.load_gather(tablev, [_splat(rb + 3)])
        npy2 = plsc.load_gather(tablev, [_splat(rb + 4)])
        npx2 = plsc.load_gather(tablev, [_splat(rb + 5)])
        npa = (npy2 - npy1) * (npx2 - npx1)

        has = gmax >= 0.0
        hasf = _splat(jnp.where(has, 1.0, 0.0).astype(jnp.float32))

        @pl.when(jnp.logical_and(cid == 0, wid == 0))
        def _():
            v = jnp.where(iota == 0, npy1,
                jnp.where(iota == 1, npx1,
                jnp.where(iota == 2, npy2, npx2))) * hasf
            plsc.store_scatter(outv, [t * 4 + iota], v, mask=iota < 4)

        return (npy1, npx1, npy2, npx2, npa, *new_scores)

    lax.fori_loop(0, MAX_OUT, body, (zero, zero, zero, zero, zero, *scores0),
                  unroll=False)

    @pl.when(jnp.logical_and(cid == 0, wid == 0))
    def _():
        pltpu.sync_copy(outv.at[pl.ds(0, MAX_OUT * 4)], outh)


@jax.jit
def _nms(y1, x1, y2, x2, s):
    mesh = plsc.VectorSubcoreMesh(core_axis_name="c", subcore_axis_name="s",
                                  num_cores=1)
    f = functools.partial(
        pl.kernel,
        mesh=mesh,
        compiler_params=pltpu.CompilerParams(needs_layout_passes=False),
        out_type=jax.ShapeDtypeStruct((MAX_OUT * 4,), jnp.float32),
        scratch_types=[
            pltpu.VMEM((PER_W,), jnp.float32),
            pltpu.VMEM((PER_W,), jnp.float32),
            pltpu.VMEM((PER_W,), jnp.float32),
            pltpu.VMEM((PER_W,), jnp.float32),
            pltpu.VMEM((PER_W,), jnp.float32),
            pltpu.VMEM((PER_W,), jnp.float32),
            pltpu.VMEM((16,), jnp.float32),
            pltpu.VMEM_SHARED((2 * 16 * ROW,), jnp.float32),
            pltpu.VMEM((16 * ROW,), jnp.float32),
            pltpu.VMEM((MAX_OUT * 4 + 16,), jnp.float32),
        ],
    )(_nms_body)
    return f(y1, x1, y2, x2, s)


def kernel(boxes, scores, max_output_size):
    n = boxes.shape[0]
    pad = N_PAD - n
    y1 = jnp.pad(boxes[:, 0], (0, pad))
    x1 = jnp.pad(boxes[:, 1], (0, pad))
    y2 = jnp.pad(boxes[:, 2], (0, pad))
    x2 = jnp.pad(boxes[:, 3], (0, pad))
    s = jnp.pad(scores, (0, pad), constant_values=-1.0)
    out = _nms(y1, x1, y2, x2, s).reshape(MAX_OUT, 4)
    keep = (lax.iota(jnp.int32, MAX_OUT) < max_output_size)[:, None]
    return jnp.where(keep, out, 0.0)


# speedup vs baseline: 1.2179x; 1.0387x over previous
import functools

import jax
import jax.numpy as jnp
from jax import lax
from jax.experimental import pallas as pl
from jax.experimental.pallas import tpu as pltpu
from jax.experimental.pallas import tpu_sc as plsc

N_PAD = 5120
PER_W = N_PAD // 16
VREGS = PER_W // 16
MAX_OUT = 100
ROW = 8


def _splat(x):
    return jnp.full((16,), x)


def _nms_body(y1h, x1h, y2h, x2h, sh, outh,
              y1v, x1v, y2v, x2v, sv, areav,
              stage, table_sh, tablev, outv):
    cid = lax.axis_index("c")
    wid = lax.axis_index("s")
    base = wid * PER_W
    iota = lax.iota(jnp.int32, 16)

    pltpu.sync_copy(y1h.at[pl.ds(base, PER_W)], y1v)
    pltpu.sync_copy(x1h.at[pl.ds(base, PER_W)], x1v)
    pltpu.sync_copy(y2h.at[pl.ds(base, PER_W)], y2v)
    pltpu.sync_copy(x2h.at[pl.ds(base, PER_W)], x2v)
    pltpu.sync_copy(sh.at[pl.ds(base, PER_W)], sv)

    scores0 = []
    for j in range(VREGS):
        sl = pl.ds(j * 16, 16)
        areav[sl] = (y2v[sl] - y1v[sl]) * (x2v[sl] - x1v[sl])
        scores0.append(sv[sl])

    zero = jnp.zeros((16,), jnp.float32)

    def body(t, carry):
        py1, px1, py2, px2, pa = carry[:5]
        scores = carry[5:]

        best = jnp.full((16,), -2.0)
        bidx = jnp.zeros((16,), jnp.int32)
        idxv = base + iota
        new_scores = []
        for j in range(VREGS):
            sl = pl.ds(j * 16, 16)
            iy1 = jnp.maximum(py1, y1v[sl])
            ix1 = jnp.maximum(px1, x1v[sl])
            iy2 = jnp.minimum(py2, y2v[sl])
            ix2 = jnp.minimum(px2, x2v[sl])
            inter = jnp.maximum(iy2 - iy1, 0.0) * jnp.maximum(ix2 - ix1, 0.0)
            union = pa + areav[sl] - inter
            s = jnp.where(inter + inter > union, -1.0, scores[j])
            new_scores.append(s)
            gt = s > best
            best = jnp.where(gt, s, best)
            bidx = jnp.where(gt, idxv, bidx)
            idxv = idxv + 16

        lmax = jnp.max(best)
        lidx = jnp.min(jnp.where(best == lmax, bidx, jnp.int32(1 << 30)))

        li = _splat(lidx - base)
        wy1 = plsc.load_gather(y1v, [li])
        wx1 = plsc.load_gather(x1v, [li])
        wy2 = plsc.load_gather(y2v, [li])
        wx2 = plsc.load_gather(x2v, [li])

        gsp = _splat(lmax)
        lsp = _splat(lidx.astype(jnp.float32))
        row = jnp.where(iota == 0, gsp,
              jnp.where(iota == 1, lsp,
              jnp.where(iota == 2, wy1,
              jnp.where(iota == 3, wx1,
              jnp.where(iota == 4, wy2, wx2)))))
        stage[...] = row
        off = (t & 1) * (16 * ROW)
        pltpu.sync_copy(stage.at[pl.ds(0, ROW)],
                        table_sh.at[pl.ds(off + wid * ROW, ROW)])
        plsc.subcore_barrier()
        pltpu.sync_copy(table_sh.at[pl.ds(off, 16 * ROW)], tablev)

        col = iota * ROW
        vals = plsc.load_gather(tablev, [col])
        gmax = jnp.max(vals)
        wrow = plsc.all_reduce_ffs(vals == gmax)
        rb = (wrow * ROW) if wrow.shape == (16,) else _splat(wrow * ROW)
        npy1 = plsc.load_gather(tablev, [rb + 2])
        npx1 = plsc.load_gather(tablev, [rb + 3])
        npy2 = plsc.load_gather(tablev, [rb + 4])
        npx2 = plsc.load_gather(tablev, [rb + 5])
        npa = (npy2 - npy1) * (npx2 - npx1)

        has = gmax >= 0.0
        hasf = _splat(jnp.where(has, 1.0, 0.0).astype(jnp.float32))

        @pl.when(jnp.logical_and(cid == 0, wid == 0))
        def _():
            v = jnp.where(iota == 0, npy1,
                jnp.where(iota == 1, npx1,
                jnp.where(iota == 2, npy2, npx2))) * hasf
            plsc.store_scatter(outv, [t * 4 + iota], v, mask=iota < 4)

        return (npy1, npx1, npy2, npx2, npa, *new_scores)

    lax.fori_loop(0, MAX_OUT, body, (zero, zero, zero, zero, zero, *scores0),
                  unroll=False)

    @pl.when(jnp.logical_and(cid == 0, wid == 0))
    def _():
        pltpu.sync_copy(outv.at[pl.ds(0, MAX_OUT * 4)], outh)


@jax.jit
def _nms(y1, x1, y2, x2, s):
    mesh = plsc.VectorSubcoreMesh(core_axis_name="c", subcore_axis_name="s",
                                  num_cores=1)
    f = functools.partial(
        pl.kernel,
        mesh=mesh,
        compiler_params=pltpu.CompilerParams(needs_layout_passes=False),
        out_type=jax.ShapeDtypeStruct((MAX_OUT * 4,), jnp.float32),
        scratch_types=[
            pltpu.VMEM((PER_W,), jnp.float32),
            pltpu.VMEM((PER_W,), jnp.float32),
            pltpu.VMEM((PER_W,), jnp.float32),
            pltpu.VMEM((PER_W,), jnp.float32),
            pltpu.VMEM((PER_W,), jnp.float32),
            pltpu.VMEM((PER_W,), jnp.float32),
            pltpu.VMEM((16,), jnp.float32),
            pltpu.VMEM_SHARED((2 * 16 * ROW,), jnp.float32),
            pltpu.VMEM((16 * ROW,), jnp.float32),
            pltpu.VMEM((MAX_OUT * 4 + 16,), jnp.float32),
        ],
    )(_nms_body)
    return f(y1, x1, y2, x2, s)


def kernel(boxes, scores, max_output_size):
    n = boxes.shape[0]
    pad = N_PAD - n
    y1 = jnp.pad(boxes[:, 0], (0, pad))
    x1 = jnp.pad(boxes[:, 1], (0, pad))
    y2 = jnp.pad(boxes[:, 2], (0, pad))
    x2 = jnp.pad(boxes[:, 3], (0, pad))
    s = jnp.pad(scores, (0, pad), constant_values=-1.0)
    out = _nms(y1, x1, y2, x2, s).reshape(MAX_OUT, 4)
    keep = (lax.iota(jnp.int32, MAX_OUT) < max_output_size)[:, None]
    return jnp.where(keep, out, 0.0)


# speedup vs baseline: 1.2320x; 1.0116x over previous
import functools

import jax
import jax.numpy as jnp
from jax import lax
from jax.experimental import pallas as pl
from jax.experimental.pallas import tpu as pltpu
from jax.experimental.pallas import tpu_sc as plsc

N_PAD = 5120
PER_W = N_PAD // 16
VREGS = PER_W // 16
MAX_OUT = 100
ROW = 8


def _splat(x):
    return jnp.full((16,), x)


def _vperm(v, p):
    return lax.gather(
        v, p[:, None],
        dimension_numbers=lax.GatherDimensionNumbers(
            offset_dims=(), collapsed_slice_dims=(0,), start_index_map=(0,)),
        slice_sizes=(1,),
        mode=lax.GatherScatterMode.PROMISE_IN_BOUNDS)


def _amax_merge(m, mi, b, bi):
    take = jnp.logical_or(b > m, jnp.logical_and(b == m, bi < mi))
    return jnp.where(take, b, m), jnp.where(take, bi, mi)


def _nms_body(y1h, x1h, y2h, x2h, sh, outh,
              y1v, x1v, y2v, x2v, sv, areav,
              stage, table_sh, tablev, outv):
    cid = lax.axis_index("c")
    wid = lax.axis_index("s")
    base = wid * PER_W
    iota = lax.iota(jnp.int32, 16)

    pltpu.sync_copy(y1h.at[pl.ds(base, PER_W)], y1v)
    pltpu.sync_copy(x1h.at[pl.ds(base, PER_W)], x1v)
    pltpu.sync_copy(y2h.at[pl.ds(base, PER_W)], y2v)
    pltpu.sync_copy(x2h.at[pl.ds(base, PER_W)], x2v)
    pltpu.sync_copy(sh.at[pl.ds(base, PER_W)], sv)

    scores0 = []
    for j in range(VREGS):
        sl = pl.ds(j * 16, 16)
        areav[sl] = (y2v[sl] - y1v[sl]) * (x2v[sl] - x1v[sl])
        scores0.append(sv[sl])

    perms = [jnp.bitwise_xor(iota, jnp.int32(p)) for p in (1, 2, 4, 8)]

    zero = jnp.zeros((16,), jnp.float32)

    def body(t, carry):
        py1, px1, py2, px2, pa = carry[:5]
        scores = carry[5:]

        best = jnp.full((16,), -2.0)
        bidx = jnp.zeros((16,), jnp.int32)
        idxv = base + iota
        new_scores = []
        for j in range(VREGS):
            sl = pl.ds(j * 16, 16)
            iy1 = jnp.maximum(py1, y1v[sl])
            ix1 = jnp.maximum(px1, x1v[sl])
            iy2 = jnp.minimum(py2, y2v[sl])
            ix2 = jnp.minimum(px2, x2v[sl])
            inter = jnp.maximum(iy2 - iy1, 0.0) * jnp.maximum(ix2 - ix1, 0.0)
            union = pa + areav[sl] - inter
            s = jnp.where(inter + inter > union, -1.0, scores[j])
            new_scores.append(s)
            gt = s > best
            best = jnp.where(gt, s, best)
            bidx = jnp.where(gt, idxv, bidx)
            idxv = idxv + 16

        lm, li = best, bidx
        for p in perms:
            lm, li = _amax_merge(lm, li, _vperm(lm, p), _vperm(li, p))

        wy1 = plsc.load_gather(y1v, [li - base])
        wx1 = plsc.load_gather(x1v, [li - base])
        wy2 = plsc.load_gather(y2v, [li - base])
        wx2 = plsc.load_gather(x2v, [li - base])

        gsp = lm
        lsp = li.astype(jnp.float32)
        row = jnp.where(iota == 0, gsp,
              jnp.where(iota == 1, lsp,
              jnp.where(iota == 2, wy1,
              jnp.where(iota == 3, wx1,
              jnp.where(iota == 4, wy2, wx2)))))
        stage[...] = row
        off = (t & 1) * (16 * ROW)
        pltpu.sync_copy(stage.at[pl.ds(0, ROW)],
                        table_sh.at[pl.ds(off + wid * ROW, ROW)])
        plsc.subcore_barrier()
        pltpu.sync_copy(table_sh.at[pl.ds(off, 16 * ROW)], tablev)

        col = iota * ROW
        vals = plsc.load_gather(tablev, [col])
        gmax = jnp.max(vals)
        wrow = plsc.all_reduce_ffs(vals == gmax)
        rb = (wrow * ROW) if wrow.shape == (16,) else _splat(wrow * ROW)
        npy1 = plsc.load_gather(tablev, [rb + 2])
        npx1 = plsc.load_gather(tablev, [rb + 3])
        npy2 = plsc.load_gather(tablev, [rb + 4])
        npx2 = plsc.load_gather(tablev, [rb + 5])
        npa = (npy2 - npy1) * (npx2 - npx1)

        has = gmax >= 0.0
        hasf = _splat(jnp.where(has, 1.0, 0.0).astype(jnp.float32))

        @pl.when(jnp.logical_and(cid == 0, wid == 0))
        def _():
            v = jnp.where(iota == 0, npy1,
                jnp.where(iota == 1, npx1,
                jnp.where(iota == 2, npy2, npx2))) * hasf
            plsc.store_scatter(outv, [t * 4 + iota], v, mask=iota < 4)

        return (npy1, npx1, npy2, npx2, npa, *new_scores)

    lax.fori_loop(0, MAX_OUT, body, (zero, zero, zero, zero, zero, *scores0),
                  unroll=False)

    @pl.when(jnp.logical_and(cid == 0, wid == 0))
    def _():
        pltpu.sync_copy(outv.at[pl.ds(0, MAX_OUT * 4)], outh)


@jax.jit
def _nms(y1, x1, y2, x2, s):
    mesh = plsc.VectorSubcoreMesh(core_axis_name="c", subcore_axis_name="s",
                                  num_cores=1)
    f = functools.partial(
        pl.kernel,
        mesh=mesh,
        compiler_params=pltpu.CompilerParams(needs_layout_passes=False),
        out_type=jax.ShapeDtypeStruct((MAX_OUT * 4,), jnp.float32),
        scratch_types=[
            pltpu.VMEM((PER_W,), jnp.float32),
            pltpu.VMEM((PER_W,), jnp.float32),
            pltpu.VMEM((PER_W,), jnp.float32),
            pltpu.VMEM((PER_W,), jnp.float32),
            pltpu.VMEM((PER_W,), jnp.float32),
            pltpu.VMEM((PER_W,), jnp.float32),
            pltpu.VMEM((16,), jnp.float32),
            pltpu.VMEM_SHARED((2 * 16 * ROW,), jnp.float32),
            pltpu.VMEM((16 * ROW,), jnp.float32),
            pltpu.VMEM((MAX_OUT * 4 + 16,), jnp.float32),
        ],
    )(_nms_body)
    return f(y1, x1, y2, x2, s)


def kernel(boxes, scores, max_output_size):
    n = boxes.shape[0]
    pad = N_PAD - n
    y1 = jnp.pad(boxes[:, 0], (0, pad))
    x1 = jnp.pad(boxes[:, 1], (0, pad))
    y2 = jnp.pad(boxes[:, 2], (0, pad))
    x2 = jnp.pad(boxes[:, 3], (0, pad))
    s = jnp.pad(scores, (0, pad), constant_values=-1.0)
    out = _nms(y1, x1, y2, x2, s).reshape(MAX_OUT, 4)
    keep = (lax.iota(jnp.int32, MAX_OUT) < max_output_size)[:, None]
    return jnp.where(keep, out, 0.0)
